# baseline (speedup 1.0000x reference)
import jax
import jax.numpy as jnp
from jax import lax
from jax.experimental import pallas as pl
from jax.experimental.pallas import tpu as pltpu
from jax.experimental.pallas import tpu_sc as plsc

N = 10000
D = 128
H = D // 2
E = 320000
NC = 2
NS = 16
NW = NC * NS
K = 96
C = 105
NG = 3
NSB = 3
BODY = 3
EPT = C * K
EP = NW * EPT
RZ0 = 632
RZL = N - (NS - 1) * RZ0


def _zero_rows(buf, acc, base, nrows):
    for j in range(nrows // K):
        pltpu.sync_copy(buf, acc.at[pl.ds(base + j * K, K)])
    rem = nrows % K
    if rem:
        pltpu.sync_copy(
            buf.at[pl.ds(0, rem)], acc.at[pl.ds(base + (nrows // K) * K, rem)]
        )


def _sc_body(cols_hbm, rows_hbm, vals_hbm, x0_hbm, x1_hbm, part_hbm,
             cidx, ridx, vals_v, gbufs, sbufs, acc, gsems, ssems):
    c = lax.axis_index("c")
    s = lax.axis_index("s")
    tid = c * NS + s
    base = s * RZ0

    pltpu.sync_copy(cols_hbm.at[tid], cidx)
    pltpu.sync_copy(rows_hbm.at[tid], ridx)
    pltpu.sync_copy(vals_hbm.at[tid], vals_v)

    for h in range(2):
        def zero_buf(i, _):
            for j in range(H // 16):
                sbufs[0][i, pl.ds(j * 16, 16)] = jnp.zeros((16,), jnp.float32)
            return 0
        lax.fori_loop(0, K, zero_buf, 0)

        @pl.when(s < NS - 1)
        def _zero_main():
            _zero_rows(sbufs[0], acc, base, RZ0)

        @pl.when(s == NS - 1)
        def _zero_last():
            _zero_rows(sbufs[0], acc, base, RZL)

        plsc.subcore_barrier()

        xh_hbm = x0_hbm if h == 0 else x1_hbm

        def scale(ci, gb, sb):
            def scale_g(g, _):
                vv = vals_v[ci, pl.ds(g * 16, 16)]
                for ee in range(16):
                    e = g * 16 + ee
                    v = vv[ee]
                    for j in range(H // 32):
                        u = gb[e, pl.ds(j * 16, 16)]
                        a = jax.lax.bitcast_convert_type(u << 16, jnp.float32)
                        b2 = jax.lax.bitcast_convert_type(
                            u & jnp.int32(-65536), jnp.float32
                        )
                        sb[e, pl.ds(j * 32, 16)] = a * v
                        sb[e, pl.ds(j * 32 + 16, 16)] = b2 * v
                return 0
            lax.fori_loop(0, K // 16, scale_g, 0)

        def do_chunk(cj, bg, bs):
            pltpu.make_async_copy(
                xh_hbm.at[cidx.at[cj]], gbufs[bg], gsems[bg]
            ).wait()

            @pl.when(cj >= NSB)
            def _wait_prev_scatter():
                pltpu.make_async_copy(
                    sbufs[bs], acc.at[ridx.at[cj]], ssems[bs]
                ).wait()

            scale(cj, gbufs[bg], sbufs[bs])

            @pl.when(cj + NG < C)
            def _prefetch():
                pltpu.async_copy(xh_hbm.at[cidx.at[cj + NG]], gbufs[bg], gsems[bg])

            pltpu.async_copy(sbufs[bs], acc.at[ridx.at[cj]], ssems[bs], add=True)

        for b in range(NG):
            pltpu.async_copy(xh_hbm.at[cidx.at[b]], gbufs[b], gsems[b])

        def body(i, _):
            for b in range(BODY):
                do_chunk(BODY * i + b, b % NG, b % NSB)
            return 0
        lax.fori_loop(0, C // BODY, body, 0)

        for b in range(NSB):
            pltpu.make_async_copy(sbufs[b], acc.at[ridx.at[0]], ssems[b]).wait()

        plsc.subcore_barrier()

        @pl.when(s < NS - 1)
        def _write_main():
            pltpu.sync_copy(
                acc.at[pl.ds(base, RZ0)], part_hbm.at[c, h, pl.ds(base, RZ0)]
            )

        @pl.when(s == NS - 1)
        def _write_last():
            pltpu.sync_copy(
                acc.at[pl.ds(base, RZL)], part_hbm.at[c, h, pl.ds(base, RZL)]
            )

        if h == 0:
            plsc.subcore_barrier()


def _sc_body_flat(cols_hbm, rows_hbm, vals_hbm, x0_hbm, x1_hbm, part_hbm,
                  cidx, ridx, vals_v,
                  g0, g1, g2, s0, s1, s2, acc,
                  gs0, gs1, gs2, ss0, ss1, ss2):
    _sc_body(cols_hbm, rows_hbm, vals_hbm, x0_hbm, x1_hbm, part_hbm,
             cidx, ridx, vals_v,
             (g0, g1, g2), (s0, s1, s2), acc,
             (gs0, gs1, gs2), (ss0, ss1, ss2))


def _sc_partials(cols, rows, vals, x0, x1):
    mesh = plsc.VectorSubcoreMesh(
        core_axis_name="c", subcore_axis_name="s", num_cores=NC, num_subcores=NS
    )
    gbuf = pltpu.VMEM((K, H // 2), jnp.int32)
    sbuf = pltpu.VMEM((K, H), jnp.float32)
    return pl.kernel(
        _sc_body_flat,
        out_type=jax.ShapeDtypeStruct((NC, 2, N, H), jnp.float32),
        mesh=mesh,
        compiler_params=pltpu.CompilerParams(use_tc_tiling_on_sc=False),
        scratch_types=[
            pltpu.VMEM((C, K), jnp.int32),
            pltpu.VMEM((C, K), jnp.int32),
            pltpu.VMEM((C, K), jnp.float32),
        ] + [gbuf] * NG + [sbuf] * NSB + [
            pltpu.VMEM_SHARED((N, H), jnp.float32),
        ] + [pltpu.SemaphoreType.DMA] * (NG + NSB),
    )(cols, rows, vals, x0, x1)


def _tc_body(p_ref, th_ref, o_ref):
    lx = jnp.concatenate(
        [p_ref[0, 0] + p_ref[1, 0], p_ref[0, 1] + p_ref[1, 1]], axis=-1
    )
    o_ref[...] = jnp.dot(lx, th_ref[...], preferred_element_type=jnp.float32)


def _perm_half():
    p = []
    for j in range(H // 32):
        p.extend(32 * j + 2 * i for i in range(16))
        p.extend(32 * j + 2 * i + 1 for i in range(16))
    return p


_PERM = _perm_half() + [H + q for q in _perm_half()]


def _tc_combine(part, theta):
    RB = 1000
    return pl.pallas_call(
        _tc_body,
        grid=(N // RB,),
        in_specs=[
            pl.BlockSpec((NC, 2, RB, H), lambda i: (0, 0, i, 0)),
            pl.BlockSpec((D, D), lambda i: (0, 0)),
        ],
        out_specs=pl.BlockSpec((RB, D), lambda i: (i, 0)),
        out_shape=jax.ShapeDtypeStruct((N, D), jnp.float32),
    )(part, theta[jnp.array(_PERM), :])


def kernel(L_indices, L_values, x, theta):
    pad = EP - E
    pad_idx = jnp.arange(pad, dtype=jnp.int32) % N
    rows = jnp.concatenate(
        [L_indices[0].astype(jnp.int32), pad_idx]).reshape(NW, C, K)
    cols = jnp.concatenate(
        [L_indices[1].astype(jnp.int32), pad_idx]).reshape(NW, C, K)
    vals = jnp.concatenate(
        [L_values.astype(jnp.float32), jnp.zeros((pad,), jnp.float32)]
    ).reshape(NW, C, K)
    x0 = jax.lax.bitcast_convert_type(
        x[:, :H].astype(jnp.bfloat16).reshape(N, H // 2, 2), jnp.int32)
    x1 = jax.lax.bitcast_convert_type(
        x[:, H:].astype(jnp.bfloat16).reshape(N, H // 2, 2), jnp.int32)
    part = _sc_partials(cols, rows, vals, x0, x1)
    return _tc_combine(part, theta)


# speedup vs baseline: 1.2365x; 1.2365x over previous
import jax
import jax.numpy as jnp
from jax import lax
from jax.experimental import pallas as pl
from jax.experimental.pallas import tpu as pltpu
from jax.experimental.pallas import tpu_sc as plsc

N = 10000
D = 128
E = 320000
NC = 2
NS = 16
NW = NC * NS
K = 48
C = 210
NG = 3
NSB = 2
NE = 6
BODY = 6
EPT = C * K
EP = NW * EPT
RZ0 = 632
RZL = N - (NS - 1) * RZ0


def _zero_rows(buf, acc, base, nrows):
    for j in range(nrows // K):
        pltpu.sync_copy(buf, acc.at[pl.ds(base + j * K, K)])
    rem = nrows % K
    if rem:
        pltpu.sync_copy(
            buf.at[pl.ds(0, rem)], acc.at[pl.ds(base + (nrows // K) * K, rem)]
        )


def _sc_body(edges_hbm, x_hbm, part_hbm,
             ebufs, gbufs, sbufs, acc, esems, gsems, ssems):
    c = lax.axis_index("c")
    s = lax.axis_index("s")
    tid = c * NS + s
    base = s * RZ0

    def zero_buf(i, _):
        for j in range(D // 16):
            sbufs[0][i, pl.ds(j * 16, 16)] = jnp.zeros((16,), jnp.float32)
        return 0
    lax.fori_loop(0, K, zero_buf, 0)

    @pl.when(s < NS - 1)
    def _zero_main():
        _zero_rows(sbufs[0], acc, base, RZ0)

    @pl.when(s == NS - 1)
    def _zero_last():
        _zero_rows(sbufs[0], acc, base, RZL)

    plsc.subcore_barrier()

    def scale(be, gb, sb):
        def scale_g(g, _):
            vv = jax.lax.bitcast_convert_type(
                ebufs[be][2, pl.ds(g * 16, 16)], jnp.float32
            )
            for ee in range(16):
                e = g * 16 + ee
                v = vv[ee]
                for j in range(D // 16):
                    sl = pl.ds(j * 16, 16)
                    sb[e, sl] = gb[e, sl] * v
            return 0
        lax.fori_loop(0, K // 16, scale_g, 0)

    def do_chunk(cj, be, bg, bs):
        pltpu.make_async_copy(
            x_hbm.at[ebufs[be].at[0]], gbufs[bg], gsems[bg]
        ).wait()

        @pl.when(cj >= NSB)
        def _recycle():
            pltpu.make_async_copy(
                sbufs[bs], acc.at[ebufs[be].at[1]], ssems[bs]
            ).wait()

            @pl.when(cj - NSB + NE < C)
            def _refill():
                eslot = (be - NSB) % NE
                pltpu.async_copy(
                    edges_hbm.at[tid, cj - NSB + NE],
                    ebufs[eslot], esems[eslot],
                )

        scale(be, gbufs[bg], sbufs[bs])
        pltpu.async_copy(sbufs[bs], acc.at[ebufs[be].at[1]], ssems[bs], add=True)

        @pl.when(cj + NG < C)
        def _prefetch_gather():
            eslot = (be + NG) % NE
            pltpu.make_async_copy(
                edges_hbm.at[tid, cj], ebufs[eslot], esems[eslot]
            ).wait()
            pltpu.async_copy(
                x_hbm.at[ebufs[eslot].at[0]], gbufs[bg], gsems[bg]
            )

    for ck in range(NE):
        pltpu.async_copy(edges_hbm.at[tid, ck], ebufs[ck], esems[ck])
    for ck in range(NG):
        pltpu.make_async_copy(
            edges_hbm.at[tid, ck], ebufs[ck], esems[ck]
        ).wait()
        pltpu.async_copy(x_hbm.at[ebufs[ck].at[0]], gbufs[ck], gsems[ck])

    def body(i, _):
        for b in range(BODY):
            do_chunk(BODY * i + b, b, b % NG, b % NSB)
        return 0
    lax.fori_loop(0, C // BODY, body, 0)

    for b in range(NSB):
        pltpu.make_async_copy(sbufs[b], acc.at[ebufs[0].at[1]], ssems[b]).wait()

    plsc.subcore_barrier()

    @pl.when(s < NS - 1)
    def _write_main():
        pltpu.sync_copy(acc.at[pl.ds(base, RZ0)], part_hbm.at[c, pl.ds(base, RZ0)])

    @pl.when(s == NS - 1)
    def _write_last():
        pltpu.sync_copy(acc.at[pl.ds(base, RZL)], part_hbm.at[c, pl.ds(base, RZL)])


def _sc_body_flat(edges_hbm, x_hbm, part_hbm,
                  e0, e1, e2, e3, e4, e5, g0, g1, g2, s0, s1, acc,
                  es0, es1, es2, es3, es4, es5, gs0, gs1, gs2, ss0, ss1):
    _sc_body(edges_hbm, x_hbm, part_hbm,
             (e0, e1, e2, e3, e4, e5), (g0, g1, g2), (s0, s1), acc,
             (es0, es1, es2, es3, es4, es5), (gs0, gs1, gs2), (ss0, ss1))


def _sc_partials(edges, x):
    mesh = plsc.VectorSubcoreMesh(
        core_axis_name="c", subcore_axis_name="s", num_cores=NC, num_subcores=NS
    )
    ebuf = pltpu.VMEM((3, K), jnp.int32)
    buf = pltpu.VMEM((K, D), jnp.float32)
    return pl.kernel(
        _sc_body_flat,
        out_type=jax.ShapeDtypeStruct((NC, N, D), jnp.float32),
        mesh=mesh,
        compiler_params=pltpu.CompilerParams(use_tc_tiling_on_sc=False),
        scratch_types=[ebuf] * NE + [buf] * (NG + NSB) + [
            pltpu.VMEM_SHARED((N, D), jnp.float32),
        ] + [pltpu.SemaphoreType.DMA] * (NE + NG + NSB),
    )(edges, x)


def _tc_body(p_ref, th_ref, o_ref):
    o_ref[...] = jnp.dot(
        p_ref[0] + p_ref[1], th_ref[...], preferred_element_type=jnp.float32
    )


def _tc_combine(part, theta):
    RB = 1000
    return pl.pallas_call(
        _tc_body,
        grid=(N // RB,),
        in_specs=[
            pl.BlockSpec((NC, RB, D), lambda i: (0, i, 0)),
            pl.BlockSpec((D, D), lambda i: (0, 0)),
        ],
        out_specs=pl.BlockSpec((RB, D), lambda i: (i, 0)),
        out_shape=jax.ShapeDtypeStruct((N, D), jnp.float32),
    )(part, theta)


def kernel(L_indices, L_values, x, theta):
    pad = EP - E
    pad_idx = jnp.arange(pad, dtype=jnp.int32) % N
    rows = jnp.concatenate([L_indices[0].astype(jnp.int32), pad_idx])
    cols = jnp.concatenate([L_indices[1].astype(jnp.int32), pad_idx])
    vals = jax.lax.bitcast_convert_type(
        jnp.concatenate(
            [L_values.astype(jnp.float32), jnp.zeros((pad,), jnp.float32)]
        ),
        jnp.int32,
    )
    edges = jnp.stack(
        [f.reshape(NW, C, K) for f in (cols, rows, vals)], axis=2
    )
    part = _sc_partials(edges, x)
    return _tc_combine(part, theta)


# speedup vs baseline: 1.7929x; 1.4500x over previous
import jax
import jax.numpy as jnp
from jax import lax
from jax.experimental import pallas as pl
from jax.experimental.pallas import tpu as pltpu
from jax.experimental.pallas import tpu_sc as plsc

N = 10000
D = 128
H = D // 2
E = 320000
NC = 2
NS = 16
NW = NC * NS
K = 96
C = 105
NG = 3
NSB = 3
BODY = 3
EPT = C * K
EP = NW * EPT
RZ0 = 632
RZL = N - (NS - 1) * RZ0


def _zero_rows(buf, acc, base, nrows):
    for j in range(nrows // K):
        pltpu.sync_copy(buf, acc.at[pl.ds(base + j * K, K)])
    rem = nrows % K
    if rem:
        pltpu.sync_copy(
            buf.at[pl.ds(0, rem)], acc.at[pl.ds(base + (nrows // K) * K, rem)]
        )


def _sc_body(cols_hbm, rows_hbm, vals_hbm, x0_hbm, x1_hbm, part_hbm,
             cidx, ridx, vals_v, gbufs, sbufs, acc, gsems, ssems):
    c = lax.axis_index("c")
    s = lax.axis_index("s")
    tid = c * NS + s
    base = s * RZ0

    pltpu.sync_copy(cols_hbm.at[tid], cidx)
    pltpu.sync_copy(rows_hbm.at[tid], ridx)
    pltpu.sync_copy(vals_hbm.at[tid], vals_v)

    for h in range(2):
        def zero_buf(i, _):
            for j in range(H // 16):
                gbufs[0][i, pl.ds(j * 16, 16)] = jnp.zeros((16,), jnp.float32)
            return 0
        lax.fori_loop(0, K, zero_buf, 0)

        @pl.when(s < NS - 1)
        def _zero_main():
            _zero_rows(gbufs[0], acc, base, RZ0)

        @pl.when(s == NS - 1)
        def _zero_last():
            _zero_rows(gbufs[0], acc, base, RZL)

        plsc.subcore_barrier()

        xh_hbm = x0_hbm if h == 0 else x1_hbm

        def scale(ci, gb, sb):
            def scale_g(g, _):
                vv = vals_v[ci, pl.ds(g * 16, 16)]
                for ee in range(16):
                    e = g * 16 + ee
                    v = vv[ee]
                    for j in range(H // 16):
                        sl = pl.ds(j * 16, 16)
                        sb[e, sl] = gb[e, sl] * v
                return 0
            lax.fori_loop(0, K // 16, scale_g, 0)

        def do_chunk(cj, bg, bs):
            pltpu.make_async_copy(
                xh_hbm.at[cidx.at[cj]], gbufs[bg], gsems[bg]
            ).wait()

            @pl.when(cj >= NSB)
            def _wait_prev_scatter():
                pltpu.make_async_copy(
                    sbufs[bs], acc.at[ridx.at[cj]], ssems[bs]
                ).wait()

            scale(cj, gbufs[bg], sbufs[bs])

            @pl.when(cj + NG < C)
            def _prefetch():
                pltpu.async_copy(xh_hbm.at[cidx.at[cj + NG]], gbufs[bg], gsems[bg])

            pltpu.async_copy(sbufs[bs], acc.at[ridx.at[cj]], ssems[bs], add=True)

        for b in range(NG):
            pltpu.async_copy(xh_hbm.at[cidx.at[b]], gbufs[b], gsems[b])

        def body(i, _):
            for b in range(BODY):
                do_chunk(BODY * i + b, b % NG, b % NSB)
            return 0
        lax.fori_loop(0, C // BODY, body, 0)

        for b in range(NSB):
            pltpu.make_async_copy(sbufs[b], acc.at[ridx.at[0]], ssems[b]).wait()

        plsc.subcore_barrier()

        @pl.when(s < NS - 1)
        def _write_main():
            pltpu.sync_copy(
                acc.at[pl.ds(base, RZ0)], part_hbm.at[c, h, pl.ds(base, RZ0)]
            )

        @pl.when(s == NS - 1)
        def _write_last():
            pltpu.sync_copy(
                acc.at[pl.ds(base, RZL)], part_hbm.at[c, h, pl.ds(base, RZL)]
            )

        if h == 0:
            plsc.subcore_barrier()


def _sc_body_flat(cols_hbm, rows_hbm, vals_hbm, x0_hbm, x1_hbm, part_hbm,
                  cidx, ridx, vals_v,
                  g0, g1, g2, s0, s1, s2, acc,
                  gs0, gs1, gs2, ss0, ss1, ss2):
    _sc_body(cols_hbm, rows_hbm, vals_hbm, x0_hbm, x1_hbm, part_hbm,
             cidx, ridx, vals_v,
             (g0, g1, g2), (s0, s1, s2), acc,
             (gs0, gs1, gs2), (ss0, ss1, ss2))


def _sc_partials(cols, rows, vals, x0, x1):
    mesh = plsc.VectorSubcoreMesh(
        core_axis_name="c", subcore_axis_name="s", num_cores=NC, num_subcores=NS
    )
    buf = pltpu.VMEM((K, H), jnp.float32)
    return pl.kernel(
        _sc_body_flat,
        out_type=jax.ShapeDtypeStruct((NC, 2, N, H), jnp.float32),
        mesh=mesh,
        compiler_params=pltpu.CompilerParams(use_tc_tiling_on_sc=False),
        scratch_types=[
            pltpu.VMEM((C, K), jnp.int32),
            pltpu.VMEM((C, K), jnp.int32),
            pltpu.VMEM((C, K), jnp.float32),
        ] + [buf] * (NG + NSB) + [
            pltpu.VMEM_SHARED((N, H), jnp.float32),
        ] + [pltpu.SemaphoreType.DMA] * (NG + NSB),
    )(cols, rows, vals, x0, x1)


def _tc_body(p_ref, th_ref, o_ref):
    lx = jnp.concatenate(
        [p_ref[0, 0] + p_ref[1, 0], p_ref[0, 1] + p_ref[1, 1]], axis=-1
    )
    o_ref[...] = jnp.dot(lx, th_ref[...], preferred_element_type=jnp.float32)


def _tc_combine(part, theta):
    RB = 1000
    return pl.pallas_call(
        _tc_body,
        grid=(N // RB,),
        in_specs=[
            pl.BlockSpec((NC, 2, RB, H), lambda i: (0, 0, i, 0)),
            pl.BlockSpec((D, D), lambda i: (0, 0)),
        ],
        out_specs=pl.BlockSpec((RB, D), lambda i: (i, 0)),
        out_shape=jax.ShapeDtypeStruct((N, D), jnp.float32),
    )(part, theta)


def kernel(L_indices, L_values, x, theta):
    pad = EP - E
    pad_idx = jnp.arange(pad, dtype=jnp.int32) % N
    rows = jnp.concatenate(
        [L_indices[0].astype(jnp.int32), pad_idx]).reshape(NW, C, K)
    cols = jnp.concatenate(
        [L_indices[1].astype(jnp.int32), pad_idx]).reshape(NW, C, K)
    vals = jnp.concatenate(
        [L_values.astype(jnp.float32), jnp.zeros((pad,), jnp.float32)]
    ).reshape(NW, C, K)
    x0 = x[:, :H]
    x1 = x[:, H:]
    part = _sc_partials(cols, rows, vals, x0, x1)
    return _tc_combine(part, theta)
